# baseline (speedup 1.0000x reference)
import functools

import jax
import jax.numpy as jnp
from jax import lax
from jax.experimental import pallas as pl
from jax.experimental.pallas import tpu as pltpu
from jax.experimental.pallas import tpu_sc as plsc

_NUM_ANCHORS = 3
_ANCHOR_T = 4.0
_LANES = 16


@functools.lru_cache(maxsize=None)
def _build_sc_kernel(num_layers, num_anchors, num_targets, num_workers):
    rows_per_w = num_targets // num_workers
    groups = rows_per_w // _LANES
    ncols = 7
    chunk = rows_per_w * ncols
    block = num_targets * ncols
    n_blocks = num_layers * num_anchors
    meta_len = num_layers * num_anchors * 2 + num_layers
    meta_pad = (-meta_len) % 8

    mesh = plsc.VectorSubcoreMesh(core_axis_name="c", subcore_axis_name="s")

    @functools.partial(
        pl.kernel,
        mesh=mesh,
        out_type=jax.ShapeDtypeStruct((num_layers * num_anchors * num_targets * ncols,),
                                      jnp.float32),
        scratch_types=[
            pltpu.VMEM((6, rows_per_w), jnp.float32),
            pltpu.VMEM((meta_len + meta_pad,), jnp.float32),
            pltpu.VMEM((n_blocks * chunk,), jnp.float32),
            pltpu.SemaphoreType.DMA,
        ],
        compiler_params=pltpu.CompilerParams(needs_layout_passes=False),
    )
    def sc_kernel(t_hbm, meta_hbm, out_hbm, t_v, meta_v, out_v, sem):
        wid = lax.axis_index("c") * (num_workers // 2) + lax.axis_index("s")
        base = wid * rows_per_w

        pltpu.sync_copy(meta_hbm, meta_v)
        pltpu.sync_copy(t_hbm.at[:, pl.ds(base, rows_per_w)], t_v)

        mv0 = meta_v[pl.ds(0, _LANES)]
        mv1 = meta_v[pl.ds(meta_len + meta_pad - _LANES, _LANES)]

        def mget(k):
            if k < _LANES:
                return jnp.full((_LANES,), mv0[k], jnp.float32)
            return jnp.full((_LANES,), mv1[k - (meta_len + meta_pad - _LANES)],
                            jnp.float32)

        stride_s = [mget(num_layers * num_anchors * 2 + i)
                    for i in range(num_layers)]
        law_s = [[mget((i * num_anchors + a) * 2) / stride_s[i]
                  for a in range(num_anchors)] for i in range(num_layers)]
        lah_s = [[mget((i * num_anchors + a) * 2 + 1) / stride_s[i]
                  for a in range(num_anchors)] for i in range(num_layers)]

        @pl.loop(0, groups)
        def _group(g):
            r0 = g * _LANES
            sl = pl.ds(r0, _LANES)
            n7 = (r0 * ncols) + lax.iota(jnp.int32, _LANES) * ncols
            c0 = t_v[0, sl]
            c1 = t_v[1, sl]
            c2 = t_v[2, sl]
            c3 = t_v[3, sl]
            c4 = t_v[4, sl]
            c5 = t_v[5, sl]
            for i in range(num_layers):
                stride = stride_s[i]
                s2 = c2 / stride
                s3 = c3 / stride
                s4 = c4 / stride
                s5 = c5 / stride
                for a in range(num_anchors):
                    j = i * num_anchors + a
                    rw = s4 / law_s[i][a]
                    rh = s5 / lah_s[i][a]
                    worst = jnp.maximum(jnp.maximum(rw, 1.0 / rw),
                                        jnp.maximum(rh, 1.0 / rh))
                    keep = worst < _ANCHOR_T
                    vals = (c0, c1, s2, s3, s4, s5,
                            jnp.full((_LANES,), float(a), jnp.float32))
                    for c in range(ncols):
                        v = jnp.where(keep, vals[c], 0.0)
                        plsc.store_scatter(out_v, [n7 + (j * chunk + c)], v)

        copies = []
        for j in range(n_blocks):
            copies.append(pltpu.async_copy(
                out_v.at[pl.ds(j * chunk, chunk)],
                out_hbm.at[pl.ds(j * block + wid * chunk, chunk)],
                sem))
        for d in copies:
            d.wait()

    return sc_kernel


def kernel(preds, targets, anchors, strides):
    del preds
    num_targets = targets.shape[0]
    num_layers, num_anchors = anchors.shape[0], anchors.shape[1]
    num_workers = 32
    t_t = targets[:, :6].T
    meta_len = num_layers * num_anchors * 2 + num_layers
    meta = jnp.concatenate([
        anchors.reshape(-1),
        strides.reshape(-1),
        jnp.zeros(((-meta_len) % 8,), jnp.float32),
    ])
    sc = _build_sc_kernel(num_layers, num_anchors, num_targets, num_workers)
    flat = sc(t_t, meta)
    matched = flat.reshape(num_layers, num_anchors, num_targets, 7)
    losses = jnp.zeros((3,), jnp.float32)
    return (matched, losses)


# speedup vs baseline: 1.5218x; 1.5218x over previous
import functools

import jax
import jax.numpy as jnp
from jax import lax
from jax.experimental import pallas as pl
from jax.experimental.pallas import tpu as pltpu
from jax.experimental.pallas import tpu_sc as plsc

_NUM_ANCHORS = 3
_ANCHOR_T = 4.0
_LANES = 16


@functools.lru_cache(maxsize=None)
def _build_sc_kernel(num_layers, num_anchors, num_targets, num_workers):
    rows_per_w = num_targets // num_workers
    groups = rows_per_w // _LANES
    ncols = 7
    n_blocks = num_layers * num_anchors
    meta_len = num_layers * num_anchors * 2 + num_layers

    mesh = plsc.VectorSubcoreMesh(core_axis_name="c", subcore_axis_name="s")

    @functools.partial(
        pl.kernel,
        mesh=mesh,
        out_type=jax.ShapeDtypeStruct(
            (num_layers, num_anchors, num_targets, ncols), jnp.float32),
        scratch_types=[
            pltpu.VMEM((6, rows_per_w), jnp.float32),
            pltpu.VMEM((128,), jnp.float32),
            pltpu.VMEM((2, rows_per_w, ncols), jnp.float32),
            pltpu.SemaphoreType.DMA,
            pltpu.SemaphoreType.DMA,
        ],
        compiler_params=pltpu.CompilerParams(
            use_tc_tiling_on_sc=True,
            needs_layout_passes=False,
        ),
    )
    def sc_kernel(t_hbm, meta_hbm, out_hbm, t_v, meta_v, out_v, sem0, sem1):
        wid = lax.axis_index("c") * (num_workers // 2) + lax.axis_index("s")
        base = wid * rows_per_w

        pltpu.sync_copy(meta_hbm, meta_v)
        pltpu.sync_copy(t_hbm.at[:, pl.ds(base, rows_per_w)], t_v)

        mv = meta_v[pl.ds(0, _LANES)]
        mv1 = meta_v[pl.ds(_LANES, _LANES)]

        def mget(k):
            if k < _LANES:
                return jnp.full((_LANES,), mv[k], jnp.float32)
            return jnp.full((_LANES,), mv1[k - _LANES], jnp.float32)

        stride_s = [mget(num_layers * num_anchors * 2 + i)
                    for i in range(num_layers)]
        law_s = [[mget((i * num_anchors + a) * 2) / stride_s[i]
                  for a in range(num_anchors)] for i in range(num_layers)]
        lah_s = [[mget((i * num_anchors + a) * 2 + 1) / stride_s[i]
                  for a in range(num_anchors)] for i in range(num_layers)]

        sems = (sem0, sem1)
        copies = [None, None]
        for j in range(n_blocks):
            i, a = j // num_anchors, j % num_anchors
            buf = out_v.at[j % 2]
            if copies[j % 2] is not None:
                copies[j % 2].wait()

            @pl.loop(0, groups)
            def _group(g):
                r0 = g * _LANES
                sl = pl.ds(r0, _LANES)
                n_local = r0 + lax.iota(jnp.int32, _LANES)
                stride = stride_s[i]
                s2 = t_v[2, sl] / stride
                s3 = t_v[3, sl] / stride
                s4 = t_v[4, sl] / stride
                s5 = t_v[5, sl] / stride
                rw = s4 / law_s[i][a]
                rh = s5 / lah_s[i][a]
                worst = jnp.maximum(jnp.maximum(rw, 1.0 / rw),
                                    jnp.maximum(rh, 1.0 / rh))
                keep = worst < _ANCHOR_T
                vals = (t_v[0, sl], t_v[1, sl], s2, s3, s4, s5,
                        jnp.full((_LANES,), float(a), jnp.float32))
                cvec = jnp.full((_LANES,), 0, jnp.int32)
                for c in range(ncols):
                    v = jnp.where(keep, vals[c], 0.0)
                    plsc.store_scatter(buf, [n_local, cvec + c], v)

            copies[j % 2] = pltpu.async_copy(
                buf, out_hbm.at[i, a, pl.ds(base, rows_per_w), :], sems[j % 2])
        for d in copies:
            d.wait()

    return sc_kernel


def kernel(preds, targets, anchors, strides):
    del preds
    num_targets = targets.shape[0]
    num_layers, num_anchors = anchors.shape[0], anchors.shape[1]
    num_workers = 32
    t_t = targets[:, :6].T
    meta_len = num_layers * num_anchors * 2 + num_layers
    meta = jnp.concatenate([
        anchors.reshape(-1),
        strides.reshape(-1),
        jnp.zeros((128 - meta_len,), jnp.float32),
    ])
    sc = _build_sc_kernel(num_layers, num_anchors, num_targets, num_workers)
    matched = sc(t_t, meta)
    losses = jnp.zeros((3,), jnp.float32)
    return (matched, losses)


# speedup vs baseline: 3.2890x; 2.1613x over previous
import functools

import jax
import jax.numpy as jnp
from jax import lax
from jax.experimental import pallas as pl
from jax.experimental.pallas import tpu as pltpu
from jax.experimental.pallas import tpu_sc as plsc

_ANCHOR_T = 4.0
_LANES = 16


@functools.lru_cache(maxsize=None)
def _build_sc_kernel(num_layers, num_anchors, num_targets, num_workers):
    rows_per_w = num_targets // num_workers
    groups = rows_per_w // _LANES
    ncols = 7
    n_blocks = num_layers * num_anchors

    mesh = plsc.VectorSubcoreMesh(core_axis_name="c", subcore_axis_name="s")

    @functools.partial(
        pl.kernel,
        mesh=mesh,
        out_type=jax.ShapeDtypeStruct(
            (num_layers, num_anchors, ncols, num_targets), jnp.float32),
        scratch_types=[
            pltpu.VMEM((6, rows_per_w), jnp.float32),
            pltpu.VMEM((128,), jnp.float32),
            pltpu.VMEM((n_blocks, ncols, rows_per_w), jnp.float32),
            pltpu.SemaphoreType.DMA,
        ],
        compiler_params=pltpu.CompilerParams(
            use_tc_tiling_on_sc=True,
            needs_layout_passes=False,
        ),
    )
    def sc_kernel(t_hbm, meta_hbm, out_hbm, t_v, meta_v, out_v, sem):
        wid = lax.axis_index("c") * (num_workers // 2) + lax.axis_index("s")
        base = wid * rows_per_w

        pltpu.sync_copy(meta_hbm, meta_v)
        pltpu.sync_copy(t_hbm.at[:, pl.ds(base, rows_per_w)], t_v)

        @pl.loop(0, n_blocks)
        def _block(j):
            midx = jnp.full((_LANES,), j * 4, jnp.int32)
            stride = plsc.load_gather(meta_v, [midx])
            law = plsc.load_gather(meta_v, [midx + 1]) / stride
            lah = plsc.load_gather(meta_v, [midx + 2]) / stride
            af = plsc.load_gather(meta_v, [midx + 3])

            @pl.loop(0, groups)
            def _group(g):
                sl = pl.ds(g * _LANES, _LANES)
                s2 = t_v[2, sl] / stride
                s3 = t_v[3, sl] / stride
                s4 = t_v[4, sl] / stride
                s5 = t_v[5, sl] / stride
                rw = s4 / law
                rh = s5 / lah
                worst = jnp.maximum(jnp.maximum(rw, 1.0 / rw),
                                    jnp.maximum(rh, 1.0 / rh))
                keep = worst < _ANCHOR_T
                vals = (t_v[0, sl], t_v[1, sl], s2, s3, s4, s5, af)
                for c in range(ncols):
                    out_v[j, c, sl] = jnp.where(keep, vals[c], 0.0)

        copies = []
        for j in range(n_blocks):
            i, a = j // num_anchors, j % num_anchors
            copies.append(pltpu.async_copy(
                out_v.at[j],
                out_hbm.at[i, a, :, pl.ds(base, rows_per_w)], sem))
        for d in copies:
            d.wait()

    return sc_kernel


def kernel(preds, targets, anchors, strides):
    del preds
    num_targets = targets.shape[0]
    num_layers, num_anchors = anchors.shape[0], anchors.shape[1]
    num_workers = 32
    t_t = targets[:, :6].T
    n_blocks = num_layers * num_anchors
    st_b = jnp.repeat(strides, num_anchors)[:, None]
    anc_b = anchors.reshape(n_blocks, 2)
    aidx = jnp.tile(jnp.arange(num_anchors, dtype=jnp.float32),
                    (num_layers,))[:, None]
    meta = jnp.concatenate([st_b, anc_b, aidx], axis=1).reshape(-1)
    meta = jnp.concatenate(
        [meta, jnp.zeros((128 - meta.shape[0],), jnp.float32)])
    sc = _build_sc_kernel(num_layers, num_anchors, num_targets, num_workers)
    out_t = sc(t_t, meta)
    matched = jnp.transpose(out_t, (0, 1, 3, 2))
    losses = jnp.zeros((3,), jnp.float32)
    return (matched, losses)


# speedup vs baseline: 6.9985x; 2.1279x over previous
import functools

import jax
import jax.numpy as jnp
from jax.experimental import pallas as pl
from jax.experimental.pallas import tpu as pltpu

_ANCHOR_T = 4.0
_CHUNK = 512


def _tc_body(num_layers, num_anchors, t_ref, anchors_ref, strides_ref, out_ref):
    t = t_ref[...]
    for i in range(num_layers):
        st = strides_ref[i]
        s = t[2:6, :] / st
        head = jnp.concatenate([t[0:2, :], s], axis=0)
        for a in range(num_anchors):
            law = anchors_ref[i, a, 0] / st
            lah = anchors_ref[i, a, 1] / st
            rw = s[2:3, :] / law
            rh = s[3:4, :] / lah
            worst = jnp.maximum(jnp.maximum(rw, 1.0 / rw),
                                jnp.maximum(rh, 1.0 / rh))
            keep = worst < _ANCHOR_T
            block = jnp.concatenate(
                [head, jnp.full((1, t.shape[1]), float(a), jnp.float32)],
                axis=0)
            out_ref[i, a] = jnp.where(keep, block, 0.0)


@functools.lru_cache(maxsize=None)
def _build_tc_kernel(num_layers, num_anchors, num_targets):
    ncols = 7
    grid = num_targets // _CHUNK
    return pl.pallas_call(
        functools.partial(_tc_body, num_layers, num_anchors),
        grid=(grid,),
        in_specs=[
            pl.BlockSpec((6, _CHUNK), lambda k: (0, k)),
            pl.BlockSpec(memory_space=pltpu.SMEM),
            pl.BlockSpec(memory_space=pltpu.SMEM),
        ],
        out_specs=pl.BlockSpec(
            (num_layers, num_anchors, ncols, _CHUNK),
            lambda k: (0, 0, 0, k)),
        out_shape=jax.ShapeDtypeStruct(
            (num_layers, num_anchors, ncols, num_targets), jnp.float32),
        compiler_params=pltpu.CompilerParams(
            dimension_semantics=("arbitrary",)),
    )


def kernel(preds, targets, anchors, strides):
    del preds
    num_targets = targets.shape[0]
    num_layers, num_anchors = anchors.shape[0], anchors.shape[1]
    t_t = targets[:, :6].T
    tc = _build_tc_kernel(num_layers, num_anchors, num_targets)
    out_t = tc(t_t, anchors, strides)
    matched = jnp.transpose(out_t, (0, 1, 3, 2))
    losses = jnp.zeros((3,), jnp.float32)
    return (matched, losses)


# speedup vs baseline: 13.6749x; 1.9540x over previous
import functools

import jax
import jax.numpy as jnp
from jax.experimental import pallas as pl
from jax.experimental.pallas import tpu as pltpu

_ANCHOR_T = 4.0
_CHUNK = 2048


def _tc_body(num_layers, num_anchors, t_ref, anchors_ref, strides_ref,
             out_ref, loss_ref):
    @pl.when(pl.program_id(0) == 0)
    def _():
        for c in range(3):
            loss_ref[c] = 0.0

    t = t_ref[...]
    for i in range(num_layers):
        st = strides_ref[i]
        s = t[2:6, :] / st
        head = jnp.concatenate([t[0:2, :], s], axis=0)
        for a in range(num_anchors):
            law = anchors_ref[i, a, 0] / st
            lah = anchors_ref[i, a, 1] / st
            rw = s[2:3, :] / law
            rh = s[3:4, :] / lah
            worst = jnp.maximum(jnp.maximum(rw, 1.0 / rw),
                                jnp.maximum(rh, 1.0 / rh))
            keep = worst < _ANCHOR_T
            block = jnp.concatenate(
                [head, jnp.full((1, t.shape[1]), float(a), jnp.float32)],
                axis=0)
            out_ref[i, a] = jnp.where(keep, block, 0.0)


@functools.lru_cache(maxsize=None)
def _build_tc_kernel(num_layers, num_anchors, num_targets):
    ncols = 7
    grid = num_targets // _CHUNK
    return pl.pallas_call(
        functools.partial(_tc_body, num_layers, num_anchors),
        grid=(grid,),
        in_specs=[
            pl.BlockSpec((6, _CHUNK), lambda k: (0, k)),
            pl.BlockSpec(memory_space=pltpu.SMEM),
            pl.BlockSpec(memory_space=pltpu.SMEM),
        ],
        out_specs=[
            pl.BlockSpec(
                (num_layers, num_anchors, ncols, _CHUNK),
                lambda k: (0, 0, 0, k)),
            pl.BlockSpec(memory_space=pltpu.SMEM),
        ],
        out_shape=[
            jax.ShapeDtypeStruct(
                (num_layers, num_anchors, ncols, num_targets), jnp.float32),
            jax.ShapeDtypeStruct((3,), jnp.float32),
        ],
        compiler_params=pltpu.CompilerParams(
            dimension_semantics=("arbitrary",)),
    )


def kernel(preds, targets, anchors, strides):
    del preds
    num_targets = targets.shape[0]
    num_layers, num_anchors = anchors.shape[0], anchors.shape[1]
    t_t = targets[:, :6].T
    tc = _build_tc_kernel(num_layers, num_anchors, num_targets)
    out_t, losses = tc(t_t, anchors, strides)
    matched = jnp.transpose(out_t, (0, 1, 3, 2))
    return (matched, losses)


# speedup vs baseline: 14.5341x; 1.0628x over previous
import functools

import jax
import jax.numpy as jnp
from jax.experimental import pallas as pl
from jax.experimental.pallas import tpu as pltpu

_ANCHOR_T = 4.0
_CHUNK = 4096


def _tc_body(num_layers, num_anchors, t_ref, anchors_ref, strides_ref,
             out_ref, loss_ref):
    @pl.when(pl.program_id(0) == 0)
    def _():
        for c in range(3):
            loss_ref[c] = 0.0

    t = t_ref[...]
    for i in range(num_layers):
        st = strides_ref[i]
        s = t[2:6, :] / st
        head = jnp.concatenate([t[0:2, :], s], axis=0)
        for a in range(num_anchors):
            law = anchors_ref[i, a, 0] / st
            lah = anchors_ref[i, a, 1] / st
            rw = s[2:3, :] / law
            rh = s[3:4, :] / lah
            worst = jnp.maximum(jnp.maximum(rw, 1.0 / rw),
                                jnp.maximum(rh, 1.0 / rh))
            keep = worst < _ANCHOR_T
            block = jnp.concatenate(
                [head, jnp.full((1, t.shape[1]), float(a), jnp.float32)],
                axis=0)
            out_ref[i, a] = jnp.where(keep, block, 0.0)


@functools.lru_cache(maxsize=None)
def _build_tc_kernel(num_layers, num_anchors, num_targets):
    ncols = 7
    grid = num_targets // _CHUNK
    return pl.pallas_call(
        functools.partial(_tc_body, num_layers, num_anchors),
        grid=(grid,),
        in_specs=[
            pl.BlockSpec((6, _CHUNK), lambda k: (0, k)),
            pl.BlockSpec(memory_space=pltpu.SMEM),
            pl.BlockSpec(memory_space=pltpu.SMEM),
        ],
        out_specs=[
            pl.BlockSpec(
                (num_layers, num_anchors, ncols, _CHUNK),
                lambda k: (0, 0, 0, k)),
            pl.BlockSpec(memory_space=pltpu.SMEM),
        ],
        out_shape=[
            jax.ShapeDtypeStruct(
                (num_layers, num_anchors, ncols, num_targets), jnp.float32),
            jax.ShapeDtypeStruct((3,), jnp.float32),
        ],
        compiler_params=pltpu.CompilerParams(
            dimension_semantics=("arbitrary",)),
    )


def kernel(preds, targets, anchors, strides):
    del preds
    num_targets = targets.shape[0]
    num_layers, num_anchors = anchors.shape[0], anchors.shape[1]
    t_t = targets[:, :6].T
    tc = _build_tc_kernel(num_layers, num_anchors, num_targets)
    out_t, losses = tc(t_t, anchors, strides)
    matched = jnp.transpose(out_t, (0, 1, 3, 2))
    return (matched, losses)
